# Initial kernel scaffold; baseline (speedup 1.0000x reference)
import jax
import jax.numpy as jnp
from jax.experimental import pallas as pl


def kernel(frames):
    raise NotImplementedError("write your pallas kernel here")


# speedup vs baseline: 2.5720x; 2.5720x over previous
import numpy as np
import jax
import jax.numpy as jnp
from jax.experimental import pallas as pl

_ALPHA = 4


def kernel(frames):
    B, T, C, H, W = frames.shape
    nsel = T // _ALPHA
    idx = np.linspace(0.0, T - 1, nsel).astype(np.int32)
    idx_list = [int(v) for v in idx]

    def slot_of(f):
        s = 0
        for v in idx_list:
            s = s + jnp.where(f > v, 1, 0)
        return s

    def body(x_ref, slow_ref, fast_ref):
        v = x_ref[...]
        fast_ref[...] = v
        slow_ref[...] = v

    blk = (1, 1, C, H, W)
    slow, fast = pl.pallas_call(
        body,
        grid=(B, T),
        in_specs=[pl.BlockSpec(blk, lambda b, f: (b, f, 0, 0, 0))],
        out_specs=[
            pl.BlockSpec(blk, lambda b, f: (b, slot_of(f), 0, 0, 0)),
            pl.BlockSpec(blk, lambda b, f: (b, f, 0, 0, 0)),
        ],
        out_shape=[
            jax.ShapeDtypeStruct((B, nsel, C, H, W), frames.dtype),
            jax.ShapeDtypeStruct((B, T, C, H, W), frames.dtype),
        ],
    )(frames)
    return (slow, fast)


# speedup vs baseline: 3.5851x; 1.3939x over previous
import numpy as np
import jax
import jax.numpy as jnp
from jax.experimental import pallas as pl

_ALPHA = 4


def kernel(frames):
    B, T, C, H, W = frames.shape
    nsel = T // _ALPHA
    idx = np.linspace(0.0, T - 1, nsel).astype(np.int32)
    idx_list = [int(v) for v in idx]

    def slot_of(f):
        s = 0
        for v in idx_list:
            s = s + jnp.where(f > v, 1, 0)
        return s

    def body(x_ref, slow_ref, fast_ref):
        v = x_ref[...]
        fast_ref[...] = v
        slow_ref[...] = v

    BB = 2
    blk = (BB, 1, C, H, W)
    slow, fast = pl.pallas_call(
        body,
        grid=(B // BB, T),
        in_specs=[pl.BlockSpec(blk, lambda b, f: (b, f, 0, 0, 0))],
        out_specs=[
            pl.BlockSpec(blk, lambda b, f: (b, slot_of(f), 0, 0, 0)),
            pl.BlockSpec(blk, lambda b, f: (b, f, 0, 0, 0)),
        ],
        out_shape=[
            jax.ShapeDtypeStruct((B, nsel, C, H, W), frames.dtype),
            jax.ShapeDtypeStruct((B, T, C, H, W), frames.dtype),
        ],
    )(frames)
    return (slow, fast)


# speedup vs baseline: 4.3965x; 1.2263x over previous
import numpy as np
import jax
import jax.numpy as jnp
from jax.experimental import pallas as pl

_ALPHA = 4


def kernel(frames):
    B, T, C, H, W = frames.shape
    nsel = T // _ALPHA
    idx = np.linspace(0.0, T - 1, nsel).astype(np.int32)
    idx_list = [int(v) for v in idx]

    def slot_of(f):
        s = 0
        for v in idx_list:
            s = s + jnp.where(f > v, 1, 0)
        return s

    def body(x_ref, slow_ref, fast_ref):
        v = x_ref[...]
        fast_ref[...] = v
        slow_ref[...] = v

    BB = 4
    blk = (BB, 1, C, H, W)
    slow, fast = pl.pallas_call(
        body,
        grid=(B // BB, T),
        in_specs=[pl.BlockSpec(blk, lambda b, f: (b, f, 0, 0, 0))],
        out_specs=[
            pl.BlockSpec(blk, lambda b, f: (b, slot_of(f), 0, 0, 0)),
            pl.BlockSpec(blk, lambda b, f: (b, f, 0, 0, 0)),
        ],
        out_shape=[
            jax.ShapeDtypeStruct((B, nsel, C, H, W), frames.dtype),
            jax.ShapeDtypeStruct((B, T, C, H, W), frames.dtype),
        ],
    )(frames)
    return (slow, fast)


# speedup vs baseline: 4.6410x; 1.0556x over previous
import numpy as np
import jax
import jax.numpy as jnp
from jax.experimental import pallas as pl

_ALPHA = 4


def kernel(frames):
    B, T, C, H, W = frames.shape
    nsel = T // _ALPHA
    idx = np.linspace(0.0, T - 1, nsel).astype(np.int32)
    idx_list = [int(v) for v in idx]

    def slot_of(f):
        s = 0
        for v in idx_list:
            s = s + jnp.where(f > v, 1, 0)
        return s

    def body(x_ref, slow_ref, fast_ref):
        v = x_ref[...]
        fast_ref[...] = v
        slow_ref[...] = v

    BB = 8
    blk = (BB, 1, C, H, W)
    slow, fast = pl.pallas_call(
        body,
        grid=(B // BB, T),
        in_specs=[pl.BlockSpec(blk, lambda b, f: (b, f, 0, 0, 0))],
        out_specs=[
            pl.BlockSpec(blk, lambda b, f: (b, slot_of(f), 0, 0, 0)),
            pl.BlockSpec(blk, lambda b, f: (b, f, 0, 0, 0)),
        ],
        out_shape=[
            jax.ShapeDtypeStruct((B, nsel, C, H, W), frames.dtype),
            jax.ShapeDtypeStruct((B, T, C, H, W), frames.dtype),
        ],
    )(frames)
    return (slow, fast)
